# baseline (speedup 1.0000x reference)
import functools

import jax
import jax.numpy as jnp
from jax import lax
from jax.experimental import pallas as pl
from jax.experimental.pallas import tpu as pltpu
from jax.experimental.pallas import tpu_sc as plsc

_JITTER = 0.01
_NC, _NS = 2, 16


def _lane_cumsum(x):
    w = x.shape[-1]
    sh = 1
    while sh < w:
        pad = jnp.zeros_like(x[..., :sh])
        x = x + jnp.concatenate([pad, x[..., :-sh]], axis=-1)
        sh *= 2
    return x


def _sub_cumsum(x):
    n = x.shape[0]
    sh = 1
    while sh < n:
        pad = jnp.zeros_like(x[:sh])
        x = x + jnp.concatenate([pad, x[:-sh]], axis=0)
        sh *= 2
    return x


def _router_body(hs_ref, gw_ref, logits_ref, w_ref, pos_ref, off_ref):
    hs = hs_ref[...]
    gw = gw_ref[...]
    scores = jax.lax.dot_general(hs, gw, (((1,), (1,)), ((), ())),
                                 preferred_element_type=jnp.float32)
    logits_ref[...] = scores
    neg = jnp.float32(float("-inf"))

    m1 = jnp.max(scores, axis=-1, keepdims=True)
    eq1 = scores == m1
    oh1 = eq1 & (_lane_cumsum(eq1.astype(jnp.int32)) == 1)
    fac1 = jnp.maximum(jnp.abs(scores), m1)
    msk1 = (m1 - scores) / fac1 > 2.0 * _JITTER
    mg1 = jnp.where(msk1, neg, scores)
    e1 = jnp.exp(mg1 - m1)
    p1 = (jnp.sum(jnp.where(oh1, e1, 0.0), axis=-1, keepdims=True)
          / jnp.sum(e1, axis=-1, keepdims=True))

    masked = jnp.where(oh1, neg, scores)
    m2 = jnp.max(masked, axis=-1, keepdims=True)
    eq2 = masked == m2
    oh2 = eq2 & (_lane_cumsum(eq2.astype(jnp.int32)) == 1)
    fac2 = jnp.maximum(jnp.abs(scores), m2)
    msk2 = (m2 - scores) / fac2 > 2.0 * _JITTER
    mg2 = jnp.where(msk2, neg, masked)
    e2 = jnp.exp(mg2 - m2)
    p2 = (jnp.sum(jnp.where(oh2, e2, 0.0), axis=-1, keepdims=True)
          / jnp.sum(e2, axis=-1, keepdims=True))

    w_ref[...] = jnp.concatenate([p1, p2], axis=1)

    oh = jnp.concatenate([oh1, oh2], axis=0).astype(jnp.int32)
    cum = _sub_cumsum(oh)
    counts = cum[-1:, :]
    pcnt = ((counts + 7) // 8) * 8
    pincl = _lane_cumsum(pcnt)
    pexcl = pincl - pcnt
    pos = jnp.sum((cum - 1 + pexcl) * oh, axis=-1, keepdims=True)
    pos_ref[...] = pos
    off_ref[...] = jnp.concatenate(
        [jnp.zeros((1, 1), jnp.int32), pincl,
         jnp.zeros((1, off_ref.shape[1] - counts.shape[1] - 1), jnp.int32)],
        axis=1)


def _dispatch_body(pos_hbm, hs_hbm, xs_hbm, idx_v, row_v, sem):
    c = lax.axis_index("c")
    s = lax.axis_index("s")
    wid = s * _NC + c
    base = wid * 128
    tokb = lax.rem(base, hs_hbm.shape[0])
    pltpu.sync_copy(pos_hbm.at[pl.ds(base, 128)], idx_v)
    pltpu.sync_copy(hs_hbm.at[pl.ds(tokb, 128)], row_v)
    pltpu.async_copy(row_v, xs_hbm.at[idx_v], sem).wait()


def _moe_body(off_ref, pos_ref, xs_ref, wc_ref, w1_ref, w3_ref,
              w2_ref, out_ref, ys_ref, *, blk, n_ex):
    ex = pl.program_id(0)
    start = off_ref[ex]
    end = off_ref[ex + 1]
    nblk = jax.lax.div(end - start + (blk - 1), blk)
    w1m = w1_ref[0]
    w3m = w3_ref[0]
    w2m = w2_ref[0]

    def chunk(k, carry):
        rs = pl.multiple_of(start + k * blk, 8)
        xb = xs_ref[pl.ds(rs, blk), :]
        t1 = jax.lax.dot_general(xb, w1m, (((1,), (1,)), ((), ())),
                                 preferred_element_type=jnp.float32)
        t3 = jax.lax.dot_general(xb, w3m, (((1,), (1,)), ((), ())),
                                 preferred_element_type=jnp.float32)
        h = (t1 * jax.nn.sigmoid(t1)) * t3
        yb = jax.lax.dot_general(h, w2m, (((1,), (1,)), ((), ())),
                                 preferred_element_type=jnp.float32)
        ys_ref[pl.ds(rs, blk), :] = yb
        return carry

    jax.lax.fori_loop(0, nblk, chunk, 0)

    @pl.when(ex == n_ex - 1)
    def _combine():
        n_tok = out_ref.shape[0]

        def tok(t, carry):
            p0 = pos_ref[t]
            p1 = pos_ref[n_tok + t]
            y0 = ys_ref[pl.ds(p0, 1), :]
            y1 = ys_ref[pl.ds(p1, 1), :]
            w0 = wc_ref[pl.ds(t, 1), 0:1]
            w1v = wc_ref[pl.ds(t, 1), 1:2]
            out_ref[pl.ds(t, 1), :] = w0 * y0 + w1v * y1
            return carry

        jax.lax.fori_loop(0, n_tok, tok, 0)


def _combine_body(pos_hbm, ys_hbm, out_hbm, idx0_v, idx1_v, r0_v, r1_v, sem):
    c = lax.axis_index("c")
    s = lax.axis_index("s")
    wid = s * _NC + c
    n_tok = out_hbm.shape[0]
    d = out_hbm.shape[1]
    for half in range(2):
        tk = wid * 64 + half * 32
        pltpu.sync_copy(pos_hbm.at[pl.ds(tk, 32)], idx0_v)
        pltpu.sync_copy(pos_hbm.at[pl.ds(n_tok + tk, 32)], idx1_v)
        pltpu.async_copy(ys_hbm.at[idx0_v], r0_v, sem).wait()
        pltpu.async_copy(ys_hbm.at[idx1_v], r1_v, sem).wait()

        def row(j, carry):
            def col(q, carry2):
                sl = pl.ds(q * 16, 16)
                r0_v[j, sl] = r0_v[j, sl] + r1_v[j, sl]
                return carry2
            jax.lax.fori_loop(0, d // 16, col, 0)
            return carry

        jax.lax.fori_loop(0, 32, row, 0)
        pltpu.sync_copy(r0_v, out_hbm.at[pl.ds(tk, 32)])


def kernel(hidden_states, gate_w, w1, w2, w3):
    b, s, d = hidden_states.shape
    n = b * s
    e = gate_w.shape[0]
    ffn = w1.shape[1]
    blk = 128
    npad = 2 * n + (e - 1) * 8 + blk
    hs = hidden_states.reshape(n, d)

    logits, wts, pos, offs = pl.pallas_call(
        _router_body,
        out_shape=[
            jax.ShapeDtypeStruct((n, e), jnp.float32),
            jax.ShapeDtypeStruct((n, 2), jnp.float32),
            jax.ShapeDtypeStruct((2 * n, 1), jnp.int32),
            jax.ShapeDtypeStruct((1, 128), jnp.int32),
        ],
    )(hs, gate_w)

    pos_flat = pos.reshape(2 * n)
    offsets = offs[0, : e + 1]

    mesh = plsc.VectorSubcoreMesh(core_axis_name="c", subcore_axis_name="s",
                                  num_cores=_NC, num_subcores=_NS)
    xs = pl.kernel(
        _dispatch_body,
        out_type=jax.ShapeDtypeStruct((npad, d), jnp.float32),
        mesh=mesh,
        scratch_types=[
            pltpu.VMEM((128,), jnp.int32),
            pltpu.VMEM((128, d), jnp.float32),
            pltpu.SemaphoreType.DMA,
        ],
    )(pos_flat, hs)

    grid_spec = pltpu.PrefetchScalarGridSpec(
        num_scalar_prefetch=2,
        grid=(e,),
        in_specs=[
            pl.BlockSpec((npad, d), lambda i, *_: (0, 0)),
            pl.BlockSpec((n, 2), lambda i, *_: (0, 0)),
            pl.BlockSpec((1, ffn, d), lambda i, *_: (i, 0, 0)),
            pl.BlockSpec((1, ffn, d), lambda i, *_: (i, 0, 0)),
            pl.BlockSpec((1, d, ffn), lambda i, *_: (i, 0, 0)),
        ],
        out_specs=pl.BlockSpec((n, d), lambda i, *_: (0, 0)),
        scratch_shapes=[pltpu.VMEM((npad, d), jnp.float32)],
    )
    final = pl.pallas_call(
        functools.partial(_moe_body, blk=blk, n_ex=e),
        grid_spec=grid_spec,
        out_shape=jax.ShapeDtypeStruct((n, d), jnp.float32),
        compiler_params=pltpu.CompilerParams(
            dimension_semantics=("arbitrary",),
            vmem_limit_bytes=110 * 1024 * 1024,
        ),
    )(offsets, pos_flat, xs, wts, w1, w3, w2)
    return final.reshape(b, s, d), logits


# speedup vs baseline: 1.3625x; 1.3625x over previous
import functools

import jax
import jax.numpy as jnp
from jax import lax
from jax.experimental import pallas as pl
from jax.experimental.pallas import tpu as pltpu
from jax.experimental.pallas import tpu_sc as plsc

_JITTER = 0.01
_NC, _NS = 2, 16


def _lane_cumsum(x):
    w = x.shape[-1]
    sh = 1
    while sh < w:
        pad = jnp.zeros_like(x[..., :sh])
        x = x + jnp.concatenate([pad, x[..., :-sh]], axis=-1)
        sh *= 2
    return x


def _sub_cumsum(x):
    n = x.shape[0]
    sh = 1
    while sh < n:
        pad = jnp.zeros_like(x[:sh])
        x = x + jnp.concatenate([pad, x[:-sh]], axis=0)
        sh *= 2
    return x


def _router_body(hs_ref, gw_ref, logits_ref, w_ref, pos_ref, off_ref):
    hs = hs_ref[...]
    gw = gw_ref[...]
    scores = jax.lax.dot_general(hs, gw, (((1,), (1,)), ((), ())),
                                 preferred_element_type=jnp.float32)
    logits_ref[...] = scores
    neg = jnp.float32(float("-inf"))

    m1 = jnp.max(scores, axis=-1, keepdims=True)
    eq1 = scores == m1
    oh1 = eq1 & (_lane_cumsum(eq1.astype(jnp.int32)) == 1)
    fac1 = jnp.maximum(jnp.abs(scores), m1)
    msk1 = (m1 - scores) / fac1 > 2.0 * _JITTER
    mg1 = jnp.where(msk1, neg, scores)
    e1 = jnp.exp(mg1 - m1)
    p1 = (jnp.sum(jnp.where(oh1, e1, 0.0), axis=-1, keepdims=True)
          / jnp.sum(e1, axis=-1, keepdims=True))

    masked = jnp.where(oh1, neg, scores)
    m2 = jnp.max(masked, axis=-1, keepdims=True)
    eq2 = masked == m2
    oh2 = eq2 & (_lane_cumsum(eq2.astype(jnp.int32)) == 1)
    fac2 = jnp.maximum(jnp.abs(scores), m2)
    msk2 = (m2 - scores) / fac2 > 2.0 * _JITTER
    mg2 = jnp.where(msk2, neg, masked)
    e2 = jnp.exp(mg2 - m2)
    p2 = (jnp.sum(jnp.where(oh2, e2, 0.0), axis=-1, keepdims=True)
          / jnp.sum(e2, axis=-1, keepdims=True))

    w_ref[...] = jnp.concatenate([p1, p2], axis=1)

    oh = jnp.concatenate([oh1, oh2], axis=0).astype(jnp.int32)
    cum = _sub_cumsum(oh)
    counts = cum[-1:, :]
    pcnt = ((counts + 7) // 8) * 8
    pincl = _lane_cumsum(pcnt)
    pexcl = pincl - pcnt
    pos = jnp.sum((cum - 1 + pexcl) * oh, axis=-1, keepdims=True)
    pos_ref[...] = pos
    off_ref[...] = jnp.concatenate(
        [jnp.zeros((1, 1), jnp.int32), pincl,
         jnp.zeros((1, off_ref.shape[1] - counts.shape[1] - 1), jnp.int32)],
        axis=1)


def _moe_body(off_ref, hs_ref, prow_ref, w1_ref, w3_ref, w2_ref, ys_ref, *,
              blk):
    ex = pl.program_id(0)
    start = off_ref[ex]
    nblk = jax.lax.div(off_ref[ex + 1] - start + (blk - 1), blk)
    pos0 = prow_ref[0:1, :]
    pos1 = prow_ref[1:2, :]
    hsv = hs_ref[...]
    w1m = w1_ref[0]
    w3m = w3_ref[0]
    w2m = w2_ref[0]

    def chunk(k, carry):
        rs = pl.multiple_of(start + k * blk, 8)
        rowid = rs + jax.lax.broadcasted_iota(jnp.int32, (blk, 1), 0)
        pm = (pos0 == rowid).astype(jnp.float32) + \
             (pos1 == rowid).astype(jnp.float32)
        xb = jax.lax.dot_general(pm, hsv, (((1,), (0,)), ((), ())),
                                 preferred_element_type=jnp.float32)
        t1 = jax.lax.dot_general(xb, w1m, (((1,), (1,)), ((), ())),
                                 preferred_element_type=jnp.float32)
        t3 = jax.lax.dot_general(xb, w3m, (((1,), (1,)), ((), ())),
                                 preferred_element_type=jnp.float32)
        h = (t1 * jax.nn.sigmoid(t1)) * t3
        yb = jax.lax.dot_general(h, w2m, (((1,), (1,)), ((), ())),
                                 preferred_element_type=jnp.float32)
        ys_ref[pl.ds(rs, blk), :] = yb
        return carry

    jax.lax.fori_loop(0, nblk, chunk, 0)


def _splat(vec16, j):
    idx = jnp.full((16, 1), j, jnp.int32)
    return jax.lax.gather(
        vec16, idx,
        jax.lax.GatherDimensionNumbers(
            offset_dims=(), collapsed_slice_dims=(0,), start_index_map=(0,)),
        (1,), mode=jax.lax.GatherScatterMode.PROMISE_IN_BOUNDS)


def _combine_body(pos_hbm, w_hbm, ys_hbm, out_hbm,
                  idx0_v, idx1_v, g0_v, g1_v, r0_v, r1_v, sem):
    c = lax.axis_index("c")
    s = lax.axis_index("s")
    wid = s * _NC + c
    n_tok = out_hbm.shape[0]
    d = out_hbm.shape[1]
    for half in range(2):
        tk = wid * 64 + half * 32
        pltpu.sync_copy(pos_hbm.at[pl.ds(tk, 32)], idx0_v)
        pltpu.sync_copy(pos_hbm.at[pl.ds(n_tok + tk, 32)], idx1_v)
        pltpu.sync_copy(w_hbm.at[pl.ds(tk, 32)], g0_v)
        pltpu.sync_copy(w_hbm.at[pl.ds(n_tok + tk, 32)], g1_v)
        cp0 = pltpu.async_copy(ys_hbm.at[idx0_v], r0_v, sem)
        cp1 = pltpu.async_copy(ys_hbm.at[idx1_v], r1_v, sem)
        cp0.wait()
        cp1.wait()

        def row(j, carry):
            b16 = jax.lax.div(j, 16) * 16
            jj = jax.lax.rem(j, 16)
            gs0 = _splat(g0_v[pl.ds(b16, 16)], jj)
            gs1 = _splat(g1_v[pl.ds(b16, 16)], jj)

            def col(q, carry2):
                sl = pl.ds(q * 16, 16)
                r0_v[j, sl] = gs0 * r0_v[j, sl] + gs1 * r1_v[j, sl]
                return carry2

            jax.lax.fori_loop(0, d // 16, col, 0)
            return carry

        jax.lax.fori_loop(0, 32, row, 0)
        pltpu.sync_copy(r0_v, out_hbm.at[pl.ds(tk, 32)])


def kernel(hidden_states, gate_w, w1, w2, w3):
    b, s, d = hidden_states.shape
    n = b * s
    e = gate_w.shape[0]
    ffn = w1.shape[1]
    blk = 128
    npad = 2 * n + (e - 1) * 8 + blk
    hs = hidden_states.reshape(n, d)

    logits, wts, pos, offs = pl.pallas_call(
        _router_body,
        out_shape=[
            jax.ShapeDtypeStruct((n, e), jnp.float32),
            jax.ShapeDtypeStruct((n, 2), jnp.float32),
            jax.ShapeDtypeStruct((2 * n, 1), jnp.int32),
            jax.ShapeDtypeStruct((1, 128), jnp.int32),
        ],
    )(hs, gate_w)

    pos_flat = pos.reshape(2 * n)
    w_flat = wts.T.reshape(2 * n)
    prow = pos.reshape(2, n)
    offsets = offs[0, : e + 1]

    grid_spec = pltpu.PrefetchScalarGridSpec(
        num_scalar_prefetch=1,
        grid=(e,),
        in_specs=[
            pl.BlockSpec((n, d), lambda i, *_: (0, 0)),
            pl.BlockSpec((2, n), lambda i, *_: (0, 0)),
            pl.BlockSpec((1, ffn, d), lambda i, *_: (i, 0, 0)),
            pl.BlockSpec((1, ffn, d), lambda i, *_: (i, 0, 0)),
            pl.BlockSpec((1, d, ffn), lambda i, *_: (i, 0, 0)),
        ],
        out_specs=pl.BlockSpec((npad, d), lambda i, *_: (0, 0)),
    )
    ys = pl.pallas_call(
        functools.partial(_moe_body, blk=blk),
        grid_spec=grid_spec,
        out_shape=jax.ShapeDtypeStruct((npad, d), jnp.float32),
        compiler_params=pltpu.CompilerParams(
            dimension_semantics=("arbitrary",),
            vmem_limit_bytes=110 * 1024 * 1024,
        ),
    )(offsets, hs, prow, w1, w3, w2)

    mesh = plsc.VectorSubcoreMesh(core_axis_name="c", subcore_axis_name="s",
                                  num_cores=_NC, num_subcores=_NS)
    final = pl.kernel(
        _combine_body,
        out_type=jax.ShapeDtypeStruct((n, d), jnp.float32),
        mesh=mesh,
        scratch_types=[
            pltpu.VMEM((32,), jnp.int32),
            pltpu.VMEM((32,), jnp.int32),
            pltpu.VMEM((32,), jnp.float32),
            pltpu.VMEM((32,), jnp.float32),
            pltpu.VMEM((32, d), jnp.float32),
            pltpu.VMEM((32, d), jnp.float32),
            pltpu.SemaphoreType.DMA,
        ],
    )(pos_flat, w_flat, ys)
    return final.reshape(b, s, d), logits


# speedup vs baseline: 1.3746x; 1.0089x over previous
import functools

import jax
import jax.numpy as jnp
from jax import lax
from jax.experimental import pallas as pl
from jax.experimental.pallas import tpu as pltpu
from jax.experimental.pallas import tpu_sc as plsc

_JITTER = 0.01
_NC, _NS = 2, 16


def _lane_cumsum(x):
    w = x.shape[-1]
    sh = 1
    while sh < w:
        pad = jnp.zeros_like(x[..., :sh])
        x = x + jnp.concatenate([pad, x[..., :-sh]], axis=-1)
        sh *= 2
    return x


def _sub_cumsum(x):
    n = x.shape[0]
    sh = 1
    while sh < n:
        pad = jnp.zeros_like(x[:sh])
        x = x + jnp.concatenate([pad, x[:-sh]], axis=0)
        sh *= 2
    return x


def _router_body(hs_ref, gw_ref, logits_ref, w_ref, pos_ref, off_ref):
    hs = hs_ref[...]
    gw = gw_ref[...]
    scores = jax.lax.dot_general(hs, gw, (((1,), (1,)), ((), ())),
                                 preferred_element_type=jnp.float32)
    logits_ref[...] = scores
    neg = jnp.float32(float("-inf"))

    m1 = jnp.max(scores, axis=-1, keepdims=True)
    eq1 = scores == m1
    oh1 = eq1 & (_lane_cumsum(eq1.astype(jnp.int32)) == 1)
    fac1 = jnp.maximum(jnp.abs(scores), m1)
    msk1 = (m1 - scores) / fac1 > 2.0 * _JITTER
    mg1 = jnp.where(msk1, neg, scores)
    e1 = jnp.exp(mg1 - m1)
    p1 = (jnp.sum(jnp.where(oh1, e1, 0.0), axis=-1, keepdims=True)
          / jnp.sum(e1, axis=-1, keepdims=True))

    masked = jnp.where(oh1, neg, scores)
    m2 = jnp.max(masked, axis=-1, keepdims=True)
    eq2 = masked == m2
    oh2 = eq2 & (_lane_cumsum(eq2.astype(jnp.int32)) == 1)
    fac2 = jnp.maximum(jnp.abs(scores), m2)
    msk2 = (m2 - scores) / fac2 > 2.0 * _JITTER
    mg2 = jnp.where(msk2, neg, masked)
    e2 = jnp.exp(mg2 - m2)
    p2 = (jnp.sum(jnp.where(oh2, e2, 0.0), axis=-1, keepdims=True)
          / jnp.sum(e2, axis=-1, keepdims=True))

    w_ref[...] = jnp.concatenate([p1, p2], axis=1)

    oh = jnp.concatenate([oh1, oh2], axis=0).astype(jnp.int32)
    cum = _sub_cumsum(oh)
    counts = cum[-1:, :]
    pcnt = ((counts + 7) // 8) * 8
    pincl = _lane_cumsum(pcnt)
    pexcl = pincl - pcnt
    pos = jnp.sum((cum - 1 + pexcl) * oh, axis=-1, keepdims=True)
    pos_ref[...] = pos
    off_ref[...] = jnp.concatenate(
        [jnp.zeros((1, 1), jnp.int32), pincl,
         jnp.zeros((1, off_ref.shape[1] - counts.shape[1] - 1), jnp.int32)],
        axis=1)


def _moe_body(off_ref, hs_ref, prow_ref, w1_ref, w3_ref, w2_ref, ys_ref, *,
              blk):
    ex = pl.program_id(0)
    start = off_ref[ex]
    nblk = jax.lax.div(off_ref[ex + 1] - start + (blk - 1), blk)
    pos0 = prow_ref[0:1, :]
    pos1 = prow_ref[1:2, :]
    hsv = hs_ref[...]
    w1m = w1_ref[0]
    w3m = w3_ref[0]
    w2m = w2_ref[0]

    def chunk(k, carry):
        rs = pl.multiple_of(start + k * blk, 8)
        rowid = rs + jax.lax.broadcasted_iota(jnp.int32, (blk, 1), 0)
        pm = (pos0 == rowid).astype(jnp.float32) + \
             (pos1 == rowid).astype(jnp.float32)
        xb = jax.lax.dot_general(pm, hsv, (((1,), (0,)), ((), ())),
                                 preferred_element_type=jnp.float32)
        t1 = jax.lax.dot_general(xb, w1m, (((1,), (1,)), ((), ())),
                                 preferred_element_type=jnp.float32)
        t3 = jax.lax.dot_general(xb, w3m, (((1,), (1,)), ((), ())),
                                 preferred_element_type=jnp.float32)
        h = (t1 * jax.nn.sigmoid(t1)) * t3
        yb = jax.lax.dot_general(h, w2m, (((1,), (1,)), ((), ())),
                                 preferred_element_type=jnp.float32)
        ys_ref[pl.ds(rs, blk), :] = yb
        return carry

    jax.lax.fori_loop(0, nblk, chunk, 0)


def _splat(vec16, j):
    idx = jnp.full((16, 1), j, jnp.int32)
    return jax.lax.gather(
        vec16, idx,
        jax.lax.GatherDimensionNumbers(
            offset_dims=(), collapsed_slice_dims=(0,), start_index_map=(0,)),
        (1,), mode=jax.lax.GatherScatterMode.PROMISE_IN_BOUNDS)


def _combine_body(pos_hbm, w_hbm, ys_hbm, out_hbm,
                  idx0_v, idx1_v, g0_v, g1_v, r0_v, r1_v, sem):
    c = lax.axis_index("c")
    s = lax.axis_index("s")
    wid = s * _NC + c
    n_tok = out_hbm.shape[0]
    d = out_hbm.shape[1]
    for half in range(2):
        tk = wid * 64 + half * 32
        c0 = pltpu.async_copy(pos_hbm.at[pl.ds(tk, 32)], idx0_v, sem)
        c1 = pltpu.async_copy(pos_hbm.at[pl.ds(n_tok + tk, 32)], idx1_v, sem)
        c2 = pltpu.async_copy(w_hbm.at[pl.ds(tk, 32)], g0_v, sem)
        c3 = pltpu.async_copy(w_hbm.at[pl.ds(n_tok + tk, 32)], g1_v, sem)
        c0.wait()
        c1.wait()
        c2.wait()
        c3.wait()
        cp0 = pltpu.async_copy(ys_hbm.at[idx0_v], r0_v, sem)
        cp1 = pltpu.async_copy(ys_hbm.at[idx1_v], r1_v, sem)
        cp0.wait()
        cp1.wait()

        def row(j, carry):
            b16 = jax.lax.div(j, 16) * 16
            jj = jax.lax.rem(j, 16)
            gs0 = _splat(g0_v[pl.ds(b16, 16)], jj)
            gs1 = _splat(g1_v[pl.ds(b16, 16)], jj)

            def col(q, carry2):
                sl = pl.ds(q * 16, 16)
                r0_v[j, sl] = gs0 * r0_v[j, sl] + gs1 * r1_v[j, sl]
                return carry2

            jax.lax.fori_loop(0, d // 16, col, 0)
            return carry

        jax.lax.fori_loop(0, 32, row, 0)
        pltpu.sync_copy(r0_v, out_hbm.at[pl.ds(tk, 32)])


def kernel(hidden_states, gate_w, w1, w2, w3):
    b, s, d = hidden_states.shape
    n = b * s
    e = gate_w.shape[0]
    ffn = w1.shape[1]
    blk = 128
    npad = 2 * n + (e - 1) * 8 + blk
    hs = hidden_states.reshape(n, d)

    logits, wts, pos, offs = pl.pallas_call(
        _router_body,
        out_shape=[
            jax.ShapeDtypeStruct((n, e), jnp.float32),
            jax.ShapeDtypeStruct((n, 2), jnp.float32),
            jax.ShapeDtypeStruct((2 * n, 1), jnp.int32),
            jax.ShapeDtypeStruct((1, 128), jnp.int32),
        ],
    )(hs, gate_w)

    pos_flat = pos.reshape(2 * n)
    w_flat = wts.T.reshape(2 * n)
    prow = pos.reshape(2, n)
    offsets = offs[0, : e + 1]

    grid_spec = pltpu.PrefetchScalarGridSpec(
        num_scalar_prefetch=1,
        grid=(e,),
        in_specs=[
            pl.BlockSpec((n, d), lambda i, *_: (0, 0)),
            pl.BlockSpec((2, n), lambda i, *_: (0, 0)),
            pl.BlockSpec((1, ffn, d), lambda i, *_: (i, 0, 0)),
            pl.BlockSpec((1, ffn, d), lambda i, *_: (i, 0, 0)),
            pl.BlockSpec((1, d, ffn), lambda i, *_: (i, 0, 0)),
        ],
        out_specs=pl.BlockSpec((npad, d), lambda i, *_: (0, 0)),
    )
    ys = pl.pallas_call(
        functools.partial(_moe_body, blk=blk),
        grid_spec=grid_spec,
        out_shape=jax.ShapeDtypeStruct((npad, d), jnp.float32),
        compiler_params=pltpu.CompilerParams(
            dimension_semantics=("arbitrary",),
            vmem_limit_bytes=110 * 1024 * 1024,
        ),
    )(offsets, hs, prow, w1, w3, w2)

    mesh = plsc.VectorSubcoreMesh(core_axis_name="c", subcore_axis_name="s",
                                  num_cores=_NC, num_subcores=_NS)
    final = pl.kernel(
        _combine_body,
        out_type=jax.ShapeDtypeStruct((n, d), jnp.float32),
        mesh=mesh,
        scratch_types=[
            pltpu.VMEM((32,), jnp.int32),
            pltpu.VMEM((32,), jnp.int32),
            pltpu.VMEM((32,), jnp.float32),
            pltpu.VMEM((32,), jnp.float32),
            pltpu.VMEM((32, d), jnp.float32),
            pltpu.VMEM((32, d), jnp.float32),
            pltpu.SemaphoreType.DMA,
        ],
    )(pos_flat, w_flat, ys)
    return final.reshape(b, s, d), logits
